# baseline (speedup 1.0000x reference)
import functools

import numpy as np

import jax
import jax.numpy as jnp
from jax import lax
from jax.experimental import pallas as pl
from jax.experimental.pallas import tpu as pltpu

_CONV_CH = (16, 32, 32, 64)


def _rup(a, b):
    return -(-a // b) * b


class _Lay:
    def __init__(self, N, D, H, W, F):
        self.N, self.D, self.H, self.W, self.F = N, D, H, W, F
        self.Dp, self.Hp, self.Wp = D + 2, H + 2, W + 2
        self.cp = 128 // F
        self.Mp = N * self.Dp * self.Hp * self.Wp
        G = self.Hp * self.Wp + self.Wp + 1
        self.hal = _rup(G, 128)
        self.offs = tuple(dz * self.Hp * self.Wp + dy * self.Wp + dx
                          for dz in (-1, 0, 1) for dy in (-1, 0, 1)
                          for dx in (-1, 0, 1))
        qset = set()
        for off in self.offs:
            for p in range(F):
                qset.add((p + off) // F)
        self.qs = tuple(sorted(qset))
        self.K = len(self.qs) * 128
        cap = (12 * 1024 * 1024) * F // (2 * self.K)
        k = max(1, min(cap // self.hal, 13440 // self.hal,
                       max(1, self.Mp // (2 * self.hal))))
        self.tm = k * self.hal
        self.T = -(-self.Mp // self.tm)
        self.Mp_c = self.T * self.tm
        self.front = self.tm
        self.L = self.Mp_c + 2 * self.tm


def _fold_weights(w, lay, cpi_w):
    F, cp, qs = lay.F, lay.cp, lay.qs
    cpo_w = w.shape[0]
    Q = len(qs)
    qpos = {q: j for j, q in enumerate(qs)}
    sel = np.zeros((27, Q, F, F), np.float32)
    for t, off in enumerate(lay.offs):
        for p in range(F):
            tot = p + off
            sel[t, qpos[tot // F], tot % F, p] = 1.0
    w_r = jnp.transpose(w.reshape(cpo_w, 27, cpi_w), (1, 2, 0))
    w5 = jnp.einsum("tjPp,tic->jPipc", jnp.asarray(sel),
                    w_r.astype(jnp.float32))
    w5 = jnp.pad(w5, ((0, 0), (0, 0), (0, cp - cpi_w),
                      (0, 0), (0, cp - cpo_w)))
    return w5.reshape(Q * 128, F * cp).astype(jnp.bfloat16)


def _fold_bias(b, lay):
    cp = lay.cp
    row = jnp.pad(b[:, 0], (0, cp - b.shape[0]))
    return jnp.tile(row, (lay.F,)).reshape(1, 128)


def _fold_mask(lay):
    m = jnp.pad(jnp.ones((lay.N, lay.D, lay.H, lay.W), jnp.float32),
                ((0, 0), (1, 1), (1, 1), (1, 1)))
    m = m.reshape(lay.Mp)
    m = jnp.pad(m, (0, lay.Mp_c - lay.Mp))
    m = jnp.repeat(m.reshape(lay.Mp_c // lay.F, lay.F), lay.cp, axis=1)
    return m.astype(jnp.bfloat16)


def _embed(vol, lay):
    c = vol.shape[-1]
    vp = jnp.pad(vol, ((0, 0), (1, 1), (1, 1), (1, 1), (0, lay.cp - c)))
    flat = vp.reshape(lay.Mp, lay.cp)
    flat = jnp.pad(flat, ((lay.front, lay.L - lay.front - lay.Mp), (0, 0)))
    return flat.reshape(lay.L // lay.F, 128).astype(jnp.bfloat16)


def _conv_body(xl_ref, xc_ref, xr_ref, w_ref, b_ref, m_ref, out_ref, *rest,
               qs, halF, zero_edges, stats):
    if stats:
        stats_ref, scr = rest
    else:
        (scr,) = rest
    tmF = xc_ref.shape[0]

    scr[:halF] = xl_ref[...]
    scr[halF:halF + tmF] = xc_ref[...]
    scr[halF + tmF:] = xr_ref[...]

    if zero_edges:
        @pl.when(pl.program_id(0) == 0)
        def _():
            scr[:halF] = jnp.zeros((halF, 128), scr.dtype)

        @pl.when(pl.program_id(0) == pl.num_programs(0) - 1)
        def _():
            scr[halF + tmF:] = jnp.zeros((halF, 128), scr.dtype)

    lhs = jnp.concatenate([scr[halF + q:halF + q + tmF, :] for q in qs],
                          axis=1)
    y = jnp.dot(lhs, w_ref[...], preferred_element_type=jnp.float32)
    y = jnp.maximum(y + b_ref[...], 0.0)
    yb = y.astype(jnp.bfloat16) * m_ref[...]
    out_ref[...] = yb

    if stats:
        yf = yb.astype(jnp.float32)
        stats_ref[0, 0:1, :] = jnp.sum(yf, axis=0, keepdims=True)
        stats_ref[0, 1:2, :] = jnp.sum(yf * yf, axis=0, keepdims=True)


def _conv(x_fold, w_big, b_row, mask, lay, *, zero_edges, stats):
    F = lay.F
    tmF, halF = lay.tm // F, lay.hal // F
    T = lay.T
    r = lay.tm // lay.hal
    K = w_big.shape[0]

    body = functools.partial(_conv_body, qs=lay.qs, halF=halF,
                             zero_edges=zero_edges, stats=stats)
    out_shape = jax.ShapeDtypeStruct((lay.L // F, 128), jnp.bfloat16)
    out_specs = pl.BlockSpec((tmF, 128), lambda i: (1 + i, 0))
    if stats:
        out_shape = (out_shape,
                     jax.ShapeDtypeStruct((T, 2, 128), jnp.float32))
        out_specs = (out_specs, pl.BlockSpec((1, 2, 128), lambda i: (i, 0, 0)))

    return pl.pallas_call(
        body,
        out_shape=out_shape,
        grid=(T,),
        in_specs=[
            pl.BlockSpec((halF, 128), lambda i: (r * (i + 1) - 1, 0)),
            pl.BlockSpec((tmF, 128), lambda i: (1 + i, 0)),
            pl.BlockSpec((halF, 128), lambda i: (r * (i + 2), 0)),
            pl.BlockSpec((K, 128), lambda i: (0, 0)),
            pl.BlockSpec((1, 128), lambda i: (0, 0)),
            pl.BlockSpec((tmF, 128), lambda i: (i, 0)),
        ],
        out_specs=out_specs,
        scratch_shapes=[pltpu.VMEM((tmF + 2 * halF, 128), jnp.bfloat16)],
        compiler_params=pltpu.CompilerParams(
            dimension_semantics=("parallel",),
            vmem_limit_bytes=56 * 1024 * 1024,
        ),
    )(x_fold, x_fold, x_fold, w_big, b_row, mask)


def _mlp_body(x_ref, w1_ref, b1_ref, w2_ref, b2_ref, w3_ref, b3_ref,
              w4_ref, b4_ref, o_ref):
    dn = (((0,), (0,)), ((), ()))
    h = x_ref[...]
    h = jnp.maximum(lax.dot_general(w1_ref[...], h, dn,
                                    preferred_element_type=jnp.float32)
                    + b1_ref[...], 0.0)
    h = jnp.maximum(lax.dot_general(w2_ref[...], h, dn,
                                    preferred_element_type=jnp.float32)
                    + b2_ref[...], 0.0)
    h = jnp.maximum(lax.dot_general(w3_ref[...], h, dn,
                                    preferred_element_type=jnp.float32)
                    + b3_ref[...], 0.0)
    z = lax.dot_general(w4_ref[...], h, dn,
                        preferred_element_type=jnp.float32) + b4_ref[...]
    o_ref[...] = jax.nn.sigmoid(z)


def _mlp_head(feats_t, params):
    args = (feats_t,
            params["fc1_w"], params["fc1_b"].T,
            params["fc2_w"], params["fc2_b"].T,
            params["fc3_w"], params["fc3_b"].T,
            params["fc4_w"], params["fc4_b"].T)
    n = feats_t.shape[1]
    n_cls = params["fc4_w"].shape[1]
    out = pl.pallas_call(
        _mlp_body,
        out_shape=jax.ShapeDtypeStruct((n_cls, n), jnp.float32),
        grid=(1,),
        in_specs=[pl.BlockSpec(a.shape, lambda i, nd=len(a.shape): (0,) * nd)
                  for a in args],
        out_specs=pl.BlockSpec((n_cls, n), lambda i: (0, 0)),
        compiler_params=pltpu.CompilerParams(
            dimension_semantics=("arbitrary",)),
    )(*args)
    return out.T


def _forward(x, params):
    N, _, D, H, W = x.shape
    folds = (8, 4, 4, 2)

    vol = jnp.transpose(x, (0, 2, 3, 4, 1))
    for blk in range(4):
        wa, ba = params[f"conv{blk}a_w"], params[f"conv{blk}a_b"]
        wb, bb = params[f"conv{blk}b_w"], params[f"conv{blk}b_b"]
        lay = _Lay(N, D, H, W, folds[blk])
        mask = _fold_mask(lay)
        x_fold = _embed(vol, lay)

        wa_big = _fold_weights(wa, lay, wa.shape[1] // 27)
        wb_big = _fold_weights(wb, lay, wb.shape[1] // 27)

        y = _conv(x_fold, wa_big, _fold_bias(ba, lay), mask, lay,
                  zero_edges=False, stats=False)
        y, st = _conv(y, wb_big, _fold_bias(bb, lay), mask, lay,
                      zero_edges=True, stats=True)

        cpo = lay.cp
        cnt = jnp.float32(N * D * H * W)
        tot = jnp.sum(st, axis=0).reshape(2, lay.F, cpo)
        s, sq = jnp.sum(tot, axis=1)
        mean = s / cnt
        var = jnp.maximum(sq / cnt - mean * mean, 0.0)
        scale = params[f"bn{blk}_g"] * lax.rsqrt(var + 1e-5)
        shift = params[f"bn{blk}_b"] - mean * scale

        fF = lay.F
        core = y[lay.front // fF:(lay.front + lay.Mp) // fF]
        core = core.reshape(lay.Mp, cpo)
        core = core.reshape(N, lay.Dp, lay.Hp, lay.Wp, cpo)[
            :, 1:-1, 1:-1, 1:-1, :]
        r8 = core.reshape(N, D // 2, 2, H // 2, 2, W // 2, 2, cpo)
        pmax = jnp.max(r8, axis=(2, 4, 6)).astype(jnp.float32)
        pmin = jnp.min(r8, axis=(2, 4, 6)).astype(jnp.float32)
        vol = jnp.where(scale >= 0, pmax, pmin) * scale + shift
        D, H, W = D // 2, H // 2, W // 2

    nch = _CONV_CH[3]
    feats_t = jnp.transpose(vol[..., :nch], (4, 1, 2, 3, 0)).reshape(-1, N)
    feats_t = feats_t.astype(jnp.float32)
    return _mlp_head(feats_t, params)


def kernel(x,
           conv0a_w, conv0a_b, conv0b_w, conv0b_b, bn0_g, bn0_b,
           conv1a_w, conv1a_b, conv1b_w, conv1b_b, bn1_g, bn1_b,
           conv2a_w, conv2a_b, conv2b_w, conv2b_b, bn2_g, bn2_b,
           conv3a_w, conv3a_b, conv3b_w, conv3b_b, bn3_g, bn3_b,
           fc1_w, fc1_b, fc2_w, fc2_b, fc3_w, fc3_b, fc4_w, fc4_b):
    params = {
        "conv0a_w": conv0a_w, "conv0a_b": conv0a_b,
        "conv0b_w": conv0b_w, "conv0b_b": conv0b_b,
        "bn0_g": bn0_g, "bn0_b": bn0_b,
        "conv1a_w": conv1a_w, "conv1a_b": conv1a_b,
        "conv1b_w": conv1b_w, "conv1b_b": conv1b_b,
        "bn1_g": bn1_g, "bn1_b": bn1_b,
        "conv2a_w": conv2a_w, "conv2a_b": conv2a_b,
        "conv2b_w": conv2b_w, "conv2b_b": conv2b_b,
        "bn2_g": bn2_g, "bn2_b": bn2_b,
        "conv3a_w": conv3a_w, "conv3a_b": conv3a_b,
        "conv3b_w": conv3b_w, "conv3b_b": conv3b_b,
        "bn3_g": bn3_g, "bn3_b": bn3_b,
        "fc1_w": fc1_w, "fc1_b": fc1_b, "fc2_w": fc2_w, "fc2_b": fc2_b,
        "fc3_w": fc3_w, "fc3_b": fc3_b, "fc4_w": fc4_w, "fc4_b": fc4_b,
    }
    return _forward(x, params)
